# baseline (speedup 1.0000x reference)
import math

import jax
import jax.numpy as jnp
from jax import lax
from jax.experimental import pallas as pl
from jax.experimental.pallas import tpu as pltpu

D = 128
H = 128
B = 16
T = 512
STEPS = T // B
TOTAL = STEPS * (B * (B + 1)) // 2


def _rnn_kernel(data_ref, h0_ref, c0_ref, wih_t_ref, whh_t_ref,
                bih_ref, bhh_ref, out_ref, h_ref, c_ref, xp_ref):
    bias = bih_ref[...] + bhh_ref[...]
    wih_t = wih_t_ref[...]

    def proj_body(i, _):
        off = pl.multiple_of(i * 64, 8)
        xp_ref[pl.ds(off, 64), :] = jnp.dot(
            data_ref[pl.ds(off, 64), :], wih_t,
            preferred_element_type=jnp.float32) + bias
        return 0

    lax.fori_loop(0, TOTAL // 64, proj_body, 0)

    whh_t = whh_t_ref[...]
    h = h0_ref[...]
    c = c0_ref[...]

    def cell(x, hb, cb):
        g = x + jnp.dot(hb, whh_t, preferred_element_type=jnp.float32)
        sig = 0.5 + 0.5 * jnp.tanh(0.5 * g[:, 0:3 * H])
        gg = jnp.tanh(g[:, 3 * H:4 * H])
        ig = sig[:, 0:H]
        fg = sig[:, H:2 * H]
        og = sig[:, 2 * H:3 * H]
        c_new = fg * cb + ig * gg
        h_new = og * jnp.tanh(c_new)
        return h_new, c_new

    for p in range(B):
        bs = p + 1
        grp = math.lcm(bs, 8) // bs
        blk = grp * bs
        ngroups = STEPS // grp
        phase_end = TOTAL - STEPS * (p * (p + 1) // 2)

        hb = h[0:bs, :]
        cb = c[0:bs, :]

        def gbody(j, carry, bs=bs, grp=grp, blk=blk, phase_end=phase_end):
            hb, cb = carry
            goff = pl.multiple_of(phase_end - (j + 1) * blk, 8)
            xg = xp_ref[pl.ds(goff, blk), :]
            outs = [None] * grp
            for m in range(grp):
                s = grp - 1 - m
                x = lax.slice(xg, (s * bs, 0), ((s + 1) * bs, 4 * H))
                hb, cb = cell(x, hb, cb)
                outs[s] = hb
            block = outs[0] if grp == 1 else jnp.concatenate(outs, axis=0)
            out_ref[pl.ds(goff, blk), :] = block
            return hb, cb

        hb, cb = lax.fori_loop(0, ngroups, gbody, (hb, cb))

        if bs < B:
            h = jnp.concatenate([hb, h[bs:B, :]], axis=0)
            c = jnp.concatenate([cb, c[bs:B, :]], axis=0)
        else:
            h, c = hb, cb

    h_ref[...] = h
    c_ref[...] = c


def kernel(data, batch_sizes, h0, c0, W_ih, W_hh, b_ih, b_hh):
    del batch_sizes
    perm = jnp.concatenate([jnp.arange(0, 2 * H), jnp.arange(3 * H, 4 * H),
                            jnp.arange(2 * H, 3 * H)])
    W_ih = W_ih[perm]
    W_hh = W_hh[perm]
    b_ih = b_ih[perm]
    b_hh = b_hh[perm]
    out, h, c = pl.pallas_call(
        _rnn_kernel,
        out_shape=(
            jax.ShapeDtypeStruct((TOTAL, H), jnp.float32),
            jax.ShapeDtypeStruct((B, H), jnp.float32),
            jax.ShapeDtypeStruct((B, H), jnp.float32),
        ),
        in_specs=[pl.BlockSpec(memory_space=pltpu.VMEM)] * 7,
        out_specs=(
            pl.BlockSpec(memory_space=pltpu.VMEM),
            pl.BlockSpec(memory_space=pltpu.VMEM),
            pl.BlockSpec(memory_space=pltpu.VMEM),
        ),
        scratch_shapes=[pltpu.VMEM((TOTAL, 4 * H), jnp.float32)],
    )(data, h0, c0, W_ih.T, W_hh.T,
      b_ih.reshape(1, 4 * H), b_hh.reshape(1, 4 * H))
    return out, h, c


# speedup vs baseline: 1.0677x; 1.0677x over previous
import math

import jax
import jax.numpy as jnp
from jax import lax
from jax.experimental import pallas as pl
from jax.experimental.pallas import tpu as pltpu

D = 128
H = 128
B = 16
T = 512
STEPS = T // B
TOTAL = STEPS * (B * (B + 1)) // 2


def _rnn_kernel(data_ref, h0_ref, c0_ref, wih_t_ref, whh_t_ref,
                bih_ref, bhh_ref, out_ref, h_ref, c_ref, xp_ref):
    bias = bih_ref[...] + bhh_ref[...]
    wih_t = wih_t_ref[...]

    def proj_body(i, _):
        off = pl.multiple_of(i * 64, 8)
        xp_ref[pl.ds(off, 64), :] = jnp.dot(
            data_ref[pl.ds(off, 64), :], wih_t,
            preferred_element_type=jnp.float32) + bias
        return 0

    lax.fori_loop(0, TOTAL // 64, proj_body, 0)

    whh_bf = whh_t_ref[...].astype(jnp.bfloat16)
    h = h0_ref[...]
    c = c0_ref[...]

    def cell(x, hb, cb):
        g = x + jnp.dot(hb.astype(jnp.bfloat16), whh_bf,
                        preferred_element_type=jnp.float32)
        ig = jax.nn.sigmoid(g[:, 0:H])
        fg = jax.nn.sigmoid(g[:, H:2 * H])
        gg = jnp.tanh(g[:, 2 * H:3 * H])
        og = jax.nn.sigmoid(g[:, 3 * H:4 * H])
        c_new = fg * cb + ig * gg
        h_new = og * jnp.tanh(c_new)
        return h_new, c_new

    for p in range(B):
        bs = p + 1
        grp = math.lcm(bs, 8) // bs
        blk = grp * bs
        ngroups = STEPS // grp
        phase_end = TOTAL - STEPS * (p * (p + 1) // 2)

        hb = h[0:bs, :]
        cb = c[0:bs, :]

        def gbody(j, carry, bs=bs, grp=grp, blk=blk, phase_end=phase_end):
            hb, cb = carry
            goff = pl.multiple_of(phase_end - (j + 1) * blk, 8)
            xg = xp_ref[pl.ds(goff, blk), :]
            outs = [None] * grp
            for m in range(grp):
                s = grp - 1 - m
                x = lax.slice(xg, (s * bs, 0), ((s + 1) * bs, 4 * H))
                hb, cb = cell(x, hb, cb)
                outs[s] = hb
            block = outs[0] if grp == 1 else jnp.concatenate(outs, axis=0)
            out_ref[pl.ds(goff, blk), :] = block
            return hb, cb

        hb, cb = lax.fori_loop(0, ngroups, gbody, (hb, cb))

        if bs < B:
            h = jnp.concatenate([hb, h[bs:B, :]], axis=0)
            c = jnp.concatenate([cb, c[bs:B, :]], axis=0)
        else:
            h, c = hb, cb

    h_ref[...] = h
    c_ref[...] = c


def kernel(data, batch_sizes, h0, c0, W_ih, W_hh, b_ih, b_hh):
    del batch_sizes
    out, h, c = pl.pallas_call(
        _rnn_kernel,
        out_shape=(
            jax.ShapeDtypeStruct((TOTAL, H), jnp.float32),
            jax.ShapeDtypeStruct((B, H), jnp.float32),
            jax.ShapeDtypeStruct((B, H), jnp.float32),
        ),
        in_specs=[pl.BlockSpec(memory_space=pltpu.VMEM)] * 7,
        out_specs=(
            pl.BlockSpec(memory_space=pltpu.VMEM),
            pl.BlockSpec(memory_space=pltpu.VMEM),
            pl.BlockSpec(memory_space=pltpu.VMEM),
        ),
        scratch_shapes=[pltpu.VMEM((TOTAL, 4 * H), jnp.float32)],
    )(data, h0, c0, W_ih.T, W_hh.T,
      b_ih.reshape(1, 4 * H), b_hh.reshape(1, 4 * H))
    return out, h, c


# speedup vs baseline: 1.1055x; 1.0354x over previous
import math

import jax
import jax.numpy as jnp
from jax import lax
from jax.experimental import pallas as pl
from jax.experimental.pallas import tpu as pltpu

D = 128
H = 128
B = 16
T = 512
STEPS = T // B
TOTAL = STEPS * (B * (B + 1)) // 2


def _rnn_kernel(data_ref, h0_ref, c0_ref, wih_t_ref, whh_t_ref,
                bih_ref, bhh_ref, out_ref, h_ref, c_ref, xp_ref):
    bias = bih_ref[...] + bhh_ref[...]
    wih_t = wih_t_ref[...]

    def proj_body(i, _):
        off = pl.multiple_of(i * 64, 8)
        xp_ref[pl.ds(off, 64), :] = jnp.dot(
            data_ref[pl.ds(off, 64), :], wih_t,
            preferred_element_type=jnp.float32) + bias
        return 0

    lax.fori_loop(0, TOTAL // 64, proj_body, 0)

    whh_t = whh_t_ref[...]
    h = h0_ref[...]
    c = c0_ref[...]

    def sig(x):
        return 0.5 + 0.5 * jnp.tanh(0.5 * x)

    def cell(x, hb, cb):
        g = x + jnp.dot(hb, whh_t, preferred_element_type=jnp.float32)
        ig = sig(g[:, 0:H])
        fg = sig(g[:, H:2 * H])
        gg = jnp.tanh(g[:, 2 * H:3 * H])
        og = sig(g[:, 3 * H:4 * H])
        c_new = fg * cb + ig * gg
        h_new = og * jnp.tanh(c_new)
        return h_new, c_new

    for p in range(B):
        bs = p + 1
        grp = math.lcm(bs, 8) // bs
        blk = grp * bs
        ngroups = STEPS // grp
        phase_end = TOTAL - STEPS * (p * (p + 1) // 2)

        hb = h[0:bs, :]
        cb = c[0:bs, :]

        def gbody(j, carry, bs=bs, grp=grp, blk=blk, phase_end=phase_end):
            hb, cb = carry
            goff = pl.multiple_of(phase_end - (j + 1) * blk, 8)
            xg = xp_ref[pl.ds(goff, blk), :]
            outs = [None] * grp
            for m in range(grp):
                s = grp - 1 - m
                x = lax.slice(xg, (s * bs, 0), ((s + 1) * bs, 4 * H))
                hb, cb = cell(x, hb, cb)
                outs[s] = hb
            block = outs[0] if grp == 1 else jnp.concatenate(outs, axis=0)
            out_ref[pl.ds(goff, blk), :] = block
            return hb, cb

        hb, cb = lax.fori_loop(0, ngroups, gbody, (hb, cb))

        if bs < B:
            h = jnp.concatenate([hb, h[bs:B, :]], axis=0)
            c = jnp.concatenate([cb, c[bs:B, :]], axis=0)
        else:
            h, c = hb, cb

    h_ref[...] = h
    c_ref[...] = c


def kernel(data, batch_sizes, h0, c0, W_ih, W_hh, b_ih, b_hh):
    del batch_sizes
    out, h, c = pl.pallas_call(
        _rnn_kernel,
        out_shape=(
            jax.ShapeDtypeStruct((TOTAL, H), jnp.float32),
            jax.ShapeDtypeStruct((B, H), jnp.float32),
            jax.ShapeDtypeStruct((B, H), jnp.float32),
        ),
        in_specs=[pl.BlockSpec(memory_space=pltpu.VMEM)] * 7,
        out_specs=(
            pl.BlockSpec(memory_space=pltpu.VMEM),
            pl.BlockSpec(memory_space=pltpu.VMEM),
            pl.BlockSpec(memory_space=pltpu.VMEM),
        ),
        scratch_shapes=[pltpu.VMEM((TOTAL, 4 * H), jnp.float32)],
    )(data, h0, c0, W_ih.T, W_hh.T,
      b_ih.reshape(1, 4 * H), b_hh.reshape(1, 4 * H))
    return out, h, c
